# Initial kernel scaffold; baseline (speedup 1.0000x reference)
import jax
import jax.numpy as jnp
from jax.experimental import pallas as pl


def kernel(word_input, word_embedding):
    raise NotImplementedError("write your pallas kernel here")


# speedup vs baseline: 1.8066x; 1.8066x over previous
import jax
import jax.numpy as jnp
from jax import lax
from jax.experimental import pallas as pl
from jax.experimental.pallas import tpu as pltpu
from jax.experimental.pallas import tpu_sc as plsc

EMB = 300
NC, NS = 2, 16
NW = NC * NS


def _main_body(table_hbm, idx_hbm, out_hbm, idx_v, a_v, b_v, sem):
    n_per_w = idx_hbm.shape[1]
    wid = lax.axis_index("s") * NC + lax.axis_index("c")
    pltpu.sync_copy(idx_hbm.at[wid], idx_v)
    b0 = wid * n_per_w

    @pl.loop(0, n_per_w)
    def _(j):
        row_ids = idx_v.at[j]
        pltpu.async_copy(table_hbm.at[row_ids, pl.ds(0, 128)], a_v, sem).wait()
        pltpu.async_copy(table_hbm.at[row_ids, pl.ds(128, 128)], b_v, sem).wait()
        pltpu.sync_copy(a_v, out_hbm.at[b0 + j, :, pl.ds(0, 128)])
        pltpu.sync_copy(b_v, out_hbm.at[b0 + j, :, pl.ds(128, 128)])


def _tail_body(tail_hbm, idx_hbm, out2_hbm, idx_v, c_v, sem):
    n_per_w = idx_hbm.shape[1]
    wid = lax.axis_index("s") * NC + lax.axis_index("c")
    pltpu.sync_copy(idx_hbm.at[wid], idx_v)
    b0 = wid * n_per_w

    @pl.loop(0, n_per_w)
    def _(j):
        pltpu.async_copy(tail_hbm.at[idx_v.at[j]], c_v, sem).wait()
        pltpu.sync_copy(c_v, out2_hbm.at[b0 + j])


def kernel(word_input, word_embedding):
    batch, seq = word_input.shape
    idx = word_input.astype(jnp.int32).reshape(NW, batch // NW, seq)
    tail = jnp.pad(word_embedding[:, 256:], ((0, 0), (0, 84)))
    mesh = plsc.VectorSubcoreMesh(core_axis_name="c", subcore_axis_name="s")

    k_main = pl.kernel(
        _main_body,
        out_type=jax.ShapeDtypeStruct((batch, seq, EMB), jnp.float32),
        mesh=mesh,
        scratch_types=[
            pltpu.VMEM((batch // NW, seq), jnp.int32),
            pltpu.VMEM((seq, 128), jnp.float32),
            pltpu.VMEM((seq, 128), jnp.float32),
            pltpu.SemaphoreType.DMA,
        ],
    )
    k_tail = pl.kernel(
        _tail_body,
        out_type=jax.ShapeDtypeStruct((batch, seq, 128), jnp.float32),
        mesh=mesh,
        scratch_types=[
            pltpu.VMEM((batch // NW, seq), jnp.int32),
            pltpu.VMEM((seq, 128), jnp.float32),
            pltpu.SemaphoreType.DMA,
        ],
    )
    out_main = k_main(word_embedding, idx)
    out2 = k_tail(tail, idx)
    return lax.dynamic_update_slice(out_main, out2[:, :, :44], (0, 0, 256))


# speedup vs baseline: 2.1090x; 1.1674x over previous
import jax
import jax.numpy as jnp
from jax import lax
from jax.experimental import pallas as pl
from jax.experimental.pallas import tpu as pltpu
from jax.experimental.pallas import tpu_sc as plsc

EMB = 300
NC, NS = 2, 16
NW = NC * NS
SEQ = 50
CHUNK = 200
PLANES = CHUNK // SEQ
NBUF = 2


def _body(table_hbm, tail_hbm, idx_hbm, out_hbm, out2_hbm,
          idx_v, a0, b0_, a1, b1_, sem0, sem1):
    n_per_w = idx_hbm.shape[0] // NW
    nchunk = n_per_w // CHUNK
    wid = lax.axis_index("s") * NC + lax.axis_index("c")
    pltpu.sync_copy(idx_hbm.at[pl.ds(wid * n_per_w, n_per_w)], idx_v)
    plane0 = wid * (nchunk * PLANES)

    def rows(chunk):
        return idx_v.at[pl.ds(chunk * CHUNK, CHUNK)]

    def ring(fire, drain, write):
        for s in range(NBUF):
            fire(s, s)

        @pl.loop(0, nchunk, step=NBUF)
        def _(t):
            for s in range(NBUF):
                chunk = t + s
                drain(s)

                @pl.when(chunk + NBUF < nchunk)
                def _():
                    fire(chunk + NBUF, s)

                write(chunk, s)

    ab = ((a0, b0_, sem0), (a1, b1_, sem1))

    def fire_ab(chunk, s):
        a_v, b_v, sem = ab[s]
        pltpu.async_copy(table_hbm.at[rows(chunk), pl.ds(0, 128)], a_v, sem)
        pltpu.async_copy(table_hbm.at[rows(chunk), pl.ds(128, 128)], b_v, sem)

    def drain_ab(s):
        a_v, b_v, sem = ab[s]
        dummy = table_hbm.at[pl.ds(0, CHUNK), pl.ds(0, 128)]
        pltpu.make_async_copy(dummy, a_v, sem).wait()
        pltpu.make_async_copy(dummy, b_v, sem).wait()

    def write_ab(chunk, s):
        a_v, b_v, _ = ab[s]
        for p in range(PLANES):
            bk = plane0 + chunk * PLANES + p
            sl = pl.ds(SEQ * p, SEQ)
            pltpu.sync_copy(a_v.at[sl], out_hbm.at[bk, :, pl.ds(0, 128)])
            pltpu.sync_copy(b_v.at[sl], out_hbm.at[bk, :, pl.ds(128, 128)])

    @pl.loop(0, nchunk)
    def _(chunk):
        fire_ab(chunk, 0)
        drain_ab(0)
        write_ab(chunk, 0)

    cb = ((a0, sem0), (a1, sem1))

    def fire_c(chunk, s):
        c_v, sem = cb[s]
        pltpu.async_copy(tail_hbm.at[rows(chunk)], c_v, sem)

    def drain_c(s):
        c_v, sem = cb[s]
        pltpu.make_async_copy(tail_hbm.at[pl.ds(0, CHUNK)], c_v, sem).wait()

    def write_c(chunk, s):
        c_v, _ = cb[s]
        for p in range(PLANES):
            bk = plane0 + chunk * PLANES + p
            pltpu.sync_copy(c_v.at[pl.ds(SEQ * p, SEQ)], out2_hbm.at[bk])

    @pl.loop(0, nchunk)
    def _(chunk):
        fire_c(chunk, 0)
        drain_c(0)
        write_c(chunk, 0)


def kernel(word_input, word_embedding):
    batch, seq = word_input.shape
    n_per_w = batch * seq // NW
    idx = word_input.astype(jnp.int32).reshape(batch * seq)
    tail = jnp.pad(word_embedding[:, 256:], ((0, 0), (0, 84)))
    mesh = plsc.VectorSubcoreMesh(core_axis_name="c", subcore_axis_name="s")

    k = pl.kernel(
        _body,
        out_type=(
            jax.ShapeDtypeStruct((batch, seq, EMB), jnp.float32),
            jax.ShapeDtypeStruct((batch, seq, 128), jnp.float32),
        ),
        mesh=mesh,
        scratch_types=[
            pltpu.VMEM((n_per_w,), jnp.int32),
            pltpu.VMEM((CHUNK, 128), jnp.float32),
            pltpu.VMEM((CHUNK, 128), jnp.float32),
            pltpu.VMEM((CHUNK, 128), jnp.float32),
            pltpu.VMEM((CHUNK, 128), jnp.float32),
            pltpu.SemaphoreType.DMA,
            pltpu.SemaphoreType.DMA,
        ],
    )
    out_main, out2 = k(word_embedding, tail, idx)
    return lax.dynamic_update_slice(out_main, out2[:, :, :44], (0, 0, 256))


# speedup vs baseline: 2.1827x; 1.0350x over previous
import jax
import jax.numpy as jnp
from jax import lax
from jax.experimental import pallas as pl
from jax.experimental.pallas import tpu as pltpu
from jax.experimental.pallas import tpu_sc as plsc

EMB = 300
NC, NS = 2, 16
NW = NC * NS
SEQ = 50
CHUNK = 200
PLANES = CHUNK // SEQ
NBUF = 2


def _body(table_hbm, tail_hbm, idx_hbm, out_hbm, out2_hbm,
          idx_v, a0, b0_, a1, b1_, sem0, sem1):
    n_per_w = idx_hbm.shape[0] // NW
    nchunk = n_per_w // CHUNK
    wid = lax.axis_index("s") * NC + lax.axis_index("c")
    pltpu.sync_copy(idx_hbm.at[pl.ds(wid * n_per_w, n_per_w)], idx_v)
    plane0 = wid * (nchunk * PLANES)

    def rows(chunk):
        return idx_v.at[pl.ds(chunk * CHUNK, CHUNK)]

    def ring(fire, drain, write):
        for s in range(NBUF):
            fire(s, s)

        @pl.loop(0, nchunk, step=NBUF)
        def _(t):
            for s in range(NBUF):
                chunk = t + s
                drain(s)
                write(chunk, s)

                @pl.when(chunk + NBUF < nchunk)
                def _():
                    fire(chunk + NBUF, s)

    ab = ((a0, b0_, sem0), (a1, b1_, sem1))

    def fire_ab(chunk, s):
        a_v, b_v, sem = ab[s]
        pltpu.async_copy(table_hbm.at[rows(chunk), pl.ds(0, 128)], a_v, sem)
        pltpu.async_copy(table_hbm.at[rows(chunk), pl.ds(128, 128)], b_v, sem)

    def drain_ab(s):
        a_v, b_v, sem = ab[s]
        dummy = table_hbm.at[pl.ds(0, CHUNK), pl.ds(0, 128)]
        pltpu.make_async_copy(dummy, a_v, sem).wait()
        pltpu.make_async_copy(dummy, b_v, sem).wait()

    def write_ab(chunk, s):
        a_v, b_v, _ = ab[s]
        for p in range(PLANES):
            bk = plane0 + chunk * PLANES + p
            sl = pl.ds(SEQ * p, SEQ)
            pltpu.sync_copy(a_v.at[sl], out_hbm.at[bk, :, pl.ds(0, 128)])
            pltpu.sync_copy(b_v.at[sl], out_hbm.at[bk, :, pl.ds(128, 128)])

    ring(fire_ab, drain_ab, write_ab)

    cb = ((a0, sem0), (a1, sem1))

    def fire_c(chunk, s):
        c_v, sem = cb[s]
        pltpu.async_copy(tail_hbm.at[rows(chunk)], c_v, sem)

    def drain_c(s):
        c_v, sem = cb[s]
        pltpu.make_async_copy(tail_hbm.at[pl.ds(0, CHUNK)], c_v, sem).wait()

    def write_c(chunk, s):
        c_v, _ = cb[s]
        for p in range(PLANES):
            bk = plane0 + chunk * PLANES + p
            pltpu.sync_copy(c_v.at[pl.ds(SEQ * p, SEQ)], out2_hbm.at[bk])

    ring(fire_c, drain_c, write_c)


def kernel(word_input, word_embedding):
    batch, seq = word_input.shape
    n_per_w = batch * seq // NW
    idx = word_input.astype(jnp.int32).reshape(batch * seq)
    tail = jnp.pad(word_embedding[:, 256:], ((0, 0), (0, 84)))
    mesh = plsc.VectorSubcoreMesh(core_axis_name="c", subcore_axis_name="s")

    k = pl.kernel(
        _body,
        out_type=(
            jax.ShapeDtypeStruct((batch, seq, EMB), jnp.float32),
            jax.ShapeDtypeStruct((batch, seq, 128), jnp.float32),
        ),
        mesh=mesh,
        scratch_types=[
            pltpu.VMEM((n_per_w,), jnp.int32),
            pltpu.VMEM((CHUNK, 128), jnp.float32),
            pltpu.VMEM((CHUNK, 128), jnp.float32),
            pltpu.VMEM((CHUNK, 128), jnp.float32),
            pltpu.VMEM((CHUNK, 128), jnp.float32),
            pltpu.SemaphoreType.DMA,
            pltpu.SemaphoreType.DMA,
        ],
    )
    out_main, out2 = k(word_embedding, tail, idx)
    return lax.dynamic_update_slice(out_main, out2[:, :, :44], (0, 0, 256))
